# baseline (speedup 1.0000x reference)
import functools

import jax
import jax.numpy as jnp
from jax import lax
from jax.experimental import pallas as pl
from jax.experimental.pallas import tpu as pltpu
from jax.experimental.pallas import tpu_sc as plsc

_N = 10000
_NP = 10240
_E = 320000
_D = 128
_NC = 2
_NS = 16
_NW = _NC * _NS
_EP = 327680
_EPW = _EP // _NW
_C = 80
_NJ = _EPW // _C
_SL = 4
_SLB = 32
_NSL = _NJ // _SLB
_NWIN = _SLB // _SL
_RPT = _NP // _NS
_ZR = 8
_CW = 16


def _sc_agg(table, src3, dst3, with_cnt):
    outs = [jax.ShapeDtypeStruct((_NC, _NP, _D), jnp.float32)]
    if with_cnt:
        outs.append(jax.ShapeDtypeStruct((_NC, _NP, _D), jnp.float32))
    scratch = [
        pltpu.VMEM((_SLB, _C), jnp.int32),
        pltpu.VMEM((_SLB, _C), jnp.int32),
        pltpu.VMEM((2, _C, _D), jnp.float32),
        pltpu.VMEM((_ZR, _D), jnp.float32),
        pltpu.VMEM_SHARED((_NP, _D), jnp.float32),
        pltpu.SemaphoreType.DMA,
        pltpu.SemaphoreType.DMA,
        pltpu.SemaphoreType.DMA,
        pltpu.SemaphoreType.DMA,
    ]

    mesh = plsc.VectorSubcoreMesh(core_axis_name="c", subcore_axis_name="s")

    @functools.partial(pl.kernel, out_type=tuple(outs), mesh=mesh,
                       scratch_types=scratch)
    def k(table_h, src_h, dst_h, *refs):
        if with_cnt:
            (out_sum, out_cnt, src_v, dst_v, rows_v, zero_v, acc_s,
             gs0, gs1, ss0, ss1) = refs
        else:
            (out_sum, src_v, dst_v, rows_v, zero_v, acc_s,
             gs0, gs1, ss0, ss1) = refs
        gsem = (gs0, gs1)
        ssem = (ss0, ss1)

        core = lax.axis_index("c")
        sub = lax.axis_index("s")
        worker = core * _NS + sub
        base = sub * _RPT

        z16 = jnp.zeros((16,), jnp.float32)
        for i in range(_ZR):
            for jj in range(_D // 16):
                zero_v[i, pl.ds(jj * 16, 16)] = z16

        def zbody(i, _):
            pltpu.sync_copy(zero_v, acc_s.at[pl.ds(base + i * _ZR, _ZR)])
            return 0
        lax.fori_loop(0, _RPT // _ZR, zbody, 0)

        plsc.subcore_barrier()

        if with_cnt:
            one16 = jnp.ones((16,), jnp.float32)
            for i in range(_C):
                for jj in range(_D // 16):
                    rows_v[0, i, pl.ds(jj * 16, 16)] = one16

            def cbody(sl, _):
                pltpu.sync_copy(dst_h.at[worker, sl], dst_v)

                def cwin(w, _):
                    descs = [
                        pltpu.async_copy(rows_v.at[0],
                                         acc_s.at[dst_v.at[w * _SL + j]],
                                         ssem[j % 2], add=True)
                        for j in range(_SL)
                    ]
                    for d in descs:
                        d.wait()
                    return 0
                lax.fori_loop(0, _NWIN, cwin, 0)
                return 0
            lax.fori_loop(0, _NSL, cbody, 0)

            plsc.subcore_barrier()
            pltpu.sync_copy(acc_s.at[pl.ds(base, _RPT)],
                            out_cnt.at[core, pl.ds(base, _RPT)])
            lax.fori_loop(0, _RPT // _ZR, zbody, 0)
            plsc.subcore_barrier()

        def ebody(sl, _):
            pltpu.sync_copy(src_h.at[worker, sl], src_v)
            pltpu.sync_copy(dst_h.at[worker, sl], dst_v)
            pltpu.async_copy(table_h.at[src_v.at[0]], rows_v.at[0], gsem[0])

            def ewin(w, _):
                b = w * _SL
                for c in range(_SL):
                    j = b + c
                    p = c % 2
                    q = (c + 1) % 2

                    @pl.when(j >= 1)
                    def _():
                        pltpu.make_async_copy(
                            rows_v.at[q], acc_s.at[dst_v.at[j - 1]],
                            ssem[q]).wait()

                    @pl.when(j + 1 < _SLB)
                    def _():
                        pltpu.async_copy(table_h.at[src_v.at[j + 1]],
                                         rows_v.at[q], gsem[q])
                    pltpu.make_async_copy(table_h.at[src_v.at[j]],
                                          rows_v.at[p], gsem[p]).wait()
                    pltpu.async_copy(rows_v.at[p], acc_s.at[dst_v.at[j]],
                                     ssem[p], add=True)
                return 0
            lax.fori_loop(0, _NWIN, ewin, 0)
            pltpu.make_async_copy(rows_v.at[1],
                                  acc_s.at[dst_v.at[_SLB - 1]],
                                  ssem[1]).wait()
            return 0
        lax.fori_loop(0, _NSL, ebody, 0)

        plsc.subcore_barrier()

        pltpu.sync_copy(acc_s.at[pl.ds(base, _RPT)],
                        out_sum.at[core, pl.ds(base, _RPT)])

    res = k(table, src3, dst3)
    return res if with_cnt else res[0]


_R = 1024


def _tc_layer(sum_p, cnt_p, xin, Wl, b, Wr):
    def body(sum_ref, cnt_ref, x_ref, wl_ref, b_ref, wr_ref, o_ref):
        s = sum_ref[0] + sum_ref[1]
        c = cnt_ref[0, :, 0:1] + cnt_ref[1, :, 0:1]
        mean = s / jnp.maximum(c, 1.0)
        h = (jnp.dot(mean, wl_ref[...], preferred_element_type=jnp.float32)
             + jnp.dot(x_ref[...], wr_ref[...],
                       preferred_element_type=jnp.float32)
             + b_ref[...])
        o_ref[...] = jnp.maximum(h, 0.0)

    return pl.pallas_call(
        body,
        grid=(_NP // _R,),
        in_specs=[
            pl.BlockSpec((2, _R, _D), lambda i: (0, i, 0)),
            pl.BlockSpec((2, _R, _D), lambda i: (0, i, 0)),
            pl.BlockSpec((_R, _D), lambda i: (i, 0)),
            pl.BlockSpec((_D, _D), lambda i: (0, 0)),
            pl.BlockSpec((1, _D), lambda i: (0, 0)),
            pl.BlockSpec((_D, _D), lambda i: (0, 0)),
        ],
        out_specs=pl.BlockSpec((_R, _D), lambda i: (i, 0)),
        out_shape=jax.ShapeDtypeStruct((_NP, _D), jnp.float32),
    )(sum_p, cnt_p, xin, Wl, b, Wr)


def _tc_layer_head(sum_p, cnt_p, hin, Wl, b, Wr, Wlin, blin):
    def body(sum_ref, cnt_ref, h_ref, wl_ref, b_ref, wr_ref, wlin_ref,
             blin_ref, o_ref):
        s = sum_ref[0] + sum_ref[1]
        c = cnt_ref[0, :, 0:1] + cnt_ref[1, :, 0:1]
        mean = s / jnp.maximum(c, 1.0)
        h = (jnp.dot(mean, wl_ref[...], preferred_element_type=jnp.float32)
             + jnp.dot(h_ref[...], wr_ref[...],
                       preferred_element_type=jnp.float32)
             + b_ref[...])
        h = jnp.maximum(h, 0.0)
        o_ref[...] = (jnp.dot(h, wlin_ref[...],
                              preferred_element_type=jnp.float32)
                      + blin_ref[0, 0])

    return pl.pallas_call(
        body,
        grid=(_NP // _R,),
        in_specs=[
            pl.BlockSpec((2, _R, _D), lambda i: (0, i, 0)),
            pl.BlockSpec((2, _R, _D), lambda i: (0, i, 0)),
            pl.BlockSpec((_R, _D), lambda i: (i, 0)),
            pl.BlockSpec((_D, _D), lambda i: (0, 0)),
            pl.BlockSpec((1, _D), lambda i: (0, 0)),
            pl.BlockSpec((_D, _D), lambda i: (0, 0)),
            pl.BlockSpec((_D, 1), lambda i: (0, 0)),
            pl.BlockSpec((1, 1), lambda i: (0, 0)),
        ],
        out_specs=pl.BlockSpec((_R, 1), lambda i: (i, 0)),
        out_shape=jax.ShapeDtypeStruct((_NP, 1), jnp.float32),
    )(sum_p, cnt_p, hin, Wl, b, Wr, Wlin, blin)


def kernel(x, edge_index, W1l, b1, W1r, W2l, b2, W2r, Wlin, blin):
    pad = jnp.full((_EP - _E,), _NP - 1, jnp.int32)
    src3 = jnp.concatenate([edge_index[0].astype(jnp.int32), pad]
                           ).reshape(_NW, _NSL, _SLB, _C)
    dst3 = jnp.concatenate([edge_index[1].astype(jnp.int32), pad]
                           ).reshape(_NW, _NSL, _SLB, _C)
    b1r = b1.reshape(1, _D)
    b2r = b2.reshape(1, _D)
    blinr = blin.reshape(1, 1)

    xp = jnp.pad(x, ((0, _NP - _N), (0, 0)))

    sum1p, cntp = _sc_agg(xp, src3, dst3, with_cnt=True)
    h1 = _tc_layer(sum1p, cntp, xp, W1l, b1r, W1r)
    sum2p = _sc_agg(h1, src3, dst3, with_cnt=False)
    y = _tc_layer_head(sum2p, cntp, h1, W2l, b2r, W2r, Wlin, blinr)
    return y[:_N, 0]


# speedup vs baseline: 2.8499x; 2.8499x over previous
import functools

import jax
import jax.numpy as jnp
from jax import lax
from jax.experimental import pallas as pl
from jax.experimental.pallas import tpu as pltpu
from jax.experimental.pallas import tpu_sc as plsc

_N = 10000
_NP = 10240
_E = 320000
_D = 128
_NC = 2
_NS = 16
_NW = _NC * _NS
_EP = 327680
_EPW = _EP // _NW
_C = 80
_NJ = _EPW // _C
_SL = 4
_SLB = 32
_NSL = _NJ // _SLB
_NWIN = _SLB // _SL
_RPT = _NP // _NS
_ZR = 8
_CW = 16


def _sc_agg(table, src3, dst3, with_cnt):
    outs = [jax.ShapeDtypeStruct((_NC, _NP, _D), jnp.float32)]
    if with_cnt:
        outs.append(jax.ShapeDtypeStruct((_NC, _NP, _D), jnp.float32))
    scratch = [
        pltpu.VMEM((_SLB, _C), jnp.int32),
        pltpu.VMEM((_SLB, _C), jnp.int32),
        pltpu.VMEM((2, _C, _D), jnp.float32),
        pltpu.VMEM((_ZR, _D), jnp.float32),
        pltpu.VMEM_SHARED((_NP, _D), jnp.float32),
        pltpu.SemaphoreType.DMA,
        pltpu.SemaphoreType.DMA,
        pltpu.SemaphoreType.DMA,
        pltpu.SemaphoreType.DMA,
    ]

    mesh = plsc.VectorSubcoreMesh(core_axis_name="c", subcore_axis_name="s")

    @functools.partial(pl.kernel, out_type=tuple(outs), mesh=mesh,
                       scratch_types=scratch)
    def k(table_h, src_h, dst_h, *refs):
        if with_cnt:
            (out_sum, out_cnt, src_v, dst_v, rows_v, zero_v, acc_s,
             gs0, gs1, ss0, ss1) = refs
        else:
            (out_sum, src_v, dst_v, rows_v, zero_v, acc_s,
             gs0, gs1, ss0, ss1) = refs
        gsem = (gs0, gs1)
        ssem = (ss0, ss1)

        core = lax.axis_index("c")
        sub = lax.axis_index("s")
        worker = core * _NS + sub
        base = sub * _RPT

        z16 = jnp.zeros((16,), jnp.float32)
        for i in range(_ZR):
            for jj in range(_D // 16):
                zero_v[i, pl.ds(jj * 16, 16)] = z16

        def zbody(i, _):
            pltpu.sync_copy(zero_v, acc_s.at[pl.ds(base + i * _ZR, _ZR)])
            return 0
        lax.fori_loop(0, _RPT // _ZR, zbody, 0)

        plsc.subcore_barrier()

        if with_cnt:
            one16 = jnp.ones((16,), jnp.float32)
            for i in range(_C):
                for jj in range(_D // 16):
                    rows_v[0, i, pl.ds(jj * 16, 16)] = one16

            def cbody(sl, _):
                pltpu.sync_copy(dst_h.at[worker, sl], dst_v)

                def cwin(w, _):
                    descs = [
                        pltpu.async_copy(rows_v.at[0],
                                         acc_s.at[dst_v.at[w * _SL + j]],
                                         ssem[j % 2], add=True)
                        for j in range(_SL)
                    ]
                    for d in descs:
                        d.wait()
                    return 0
                lax.fori_loop(0, _NWIN, cwin, 0)
                return 0
            lax.fori_loop(0, _NSL, cbody, 0)

            plsc.subcore_barrier()
            pltpu.sync_copy(acc_s.at[pl.ds(base, _RPT)],
                            out_cnt.at[core, pl.ds(base, _RPT)])
            lax.fori_loop(0, _RPT // _ZR, zbody, 0)
            plsc.subcore_barrier()

        def ebody(sl, _):
            pltpu.sync_copy(src_h.at[worker, sl], src_v)
            pltpu.sync_copy(dst_h.at[worker, sl], dst_v)
            pltpu.async_copy(table_h.at[src_v.at[0]], rows_v.at[0], gsem[0])

            def ewin(w, _):
                b = w * _SL
                for c in range(_SL):
                    j = b + c
                    p = c % 2
                    q = (c + 1) % 2

                    @pl.when(j >= 1)
                    def _():
                        pltpu.make_async_copy(
                            rows_v.at[q], acc_s.at[dst_v.at[j - 1]],
                            ssem[q]).wait()

                    @pl.when(j + 1 < _SLB)
                    def _():
                        pltpu.async_copy(table_h.at[src_v.at[j + 1]],
                                         rows_v.at[q], gsem[q])
                    pltpu.make_async_copy(table_h.at[src_v.at[j]],
                                          rows_v.at[p], gsem[p]).wait()
                    pltpu.async_copy(rows_v.at[p], acc_s.at[dst_v.at[j]],
                                     ssem[p], add=True)
                return 0
            lax.fori_loop(0, _NWIN, ewin, 0)
            pltpu.make_async_copy(rows_v.at[1],
                                  acc_s.at[dst_v.at[_SLB - 1]],
                                  ssem[1]).wait()
            return 0
        lax.fori_loop(0, _NSL, ebody, 0)

        plsc.subcore_barrier()

        pltpu.sync_copy(acc_s.at[pl.ds(base, _RPT)],
                        out_sum.at[core, pl.ds(base, _RPT)])

    res = k(table, src3, dst3)
    return res if with_cnt else res[0]


_R = 1024


def _tc_layer(sum_p, cnt_p, xin, Wl, b, Wr):
    def body(sum_ref, cnt_ref, x_ref, wl_ref, b_ref, wr_ref, o_ref):
        s = sum_ref[0] + sum_ref[1]
        c = cnt_ref[0, :, 0:1] + cnt_ref[1, :, 0:1]
        mean = s / jnp.maximum(c, 1.0)
        h = (jnp.dot(mean, wl_ref[...], preferred_element_type=jnp.float32)
             + jnp.dot(x_ref[...], wr_ref[...],
                       preferred_element_type=jnp.float32)
             + b_ref[...])
        o_ref[...] = jnp.maximum(h, 0.0)

    return pl.pallas_call(
        body,
        grid=(_NP // _R,),
        in_specs=[
            pl.BlockSpec((2, _R, _D), lambda i: (0, i, 0)),
            pl.BlockSpec((2, _R, _D), lambda i: (0, i, 0)),
            pl.BlockSpec((_R, _D), lambda i: (i, 0)),
            pl.BlockSpec((_D, _D), lambda i: (0, 0)),
            pl.BlockSpec((1, _D), lambda i: (0, 0)),
            pl.BlockSpec((_D, _D), lambda i: (0, 0)),
        ],
        out_specs=pl.BlockSpec((_R, _D), lambda i: (i, 0)),
        out_shape=jax.ShapeDtypeStruct((_NP, _D), jnp.float32),
    )(sum_p, cnt_p, xin, Wl, b, Wr)


def _tc_layer_head(sum_p, cnt_p, hin, Wl, b, Wr, Wlin, blin):
    def body(sum_ref, cnt_ref, h_ref, wl_ref, b_ref, wr_ref, wlin_ref,
             blin_ref, o_ref):
        s = sum_ref[0] + sum_ref[1]
        c = cnt_ref[0, :, 0:1] + cnt_ref[1, :, 0:1]
        mean = s / jnp.maximum(c, 1.0)
        h = (jnp.dot(mean, wl_ref[...], preferred_element_type=jnp.float32)
             + jnp.dot(h_ref[...], wr_ref[...],
                       preferred_element_type=jnp.float32)
             + b_ref[...])
        h = jnp.maximum(h, 0.0)
        o_ref[...] = (jnp.dot(h, wlin_ref[...],
                              preferred_element_type=jnp.float32)
                      + blin_ref[0, 0])

    return pl.pallas_call(
        body,
        grid=(_NP // _R,),
        in_specs=[
            pl.BlockSpec((2, _R, _D), lambda i: (0, i, 0)),
            pl.BlockSpec((2, _R, _D), lambda i: (0, i, 0)),
            pl.BlockSpec((_R, _D), lambda i: (i, 0)),
            pl.BlockSpec((_D, _D), lambda i: (0, 0)),
            pl.BlockSpec((1, _D), lambda i: (0, 0)),
            pl.BlockSpec((_D, _D), lambda i: (0, 0)),
            pl.BlockSpec((_D, 1), lambda i: (0, 0)),
            pl.BlockSpec((1, 1), lambda i: (0, 0)),
        ],
        out_specs=pl.BlockSpec((_R, 1), lambda i: (i, 0)),
        out_shape=jax.ShapeDtypeStruct((_NP, 1), jnp.float32),
    )(sum_p, cnt_p, hin, Wl, b, Wr, Wlin, blin)


def kernel(x, edge_index, W1l, b1, W1r, W2l, b2, W2r, Wlin, blin):
    pad = _N + (jnp.arange(_EP - _E, dtype=jnp.int32) % (_NP - _N))
    src3 = jnp.concatenate([edge_index[0].astype(jnp.int32), pad]
                           ).reshape(_NW, _NSL, _SLB, _C)
    dst3 = jnp.concatenate([edge_index[1].astype(jnp.int32), pad]
                           ).reshape(_NW, _NSL, _SLB, _C)
    b1r = b1.reshape(1, _D)
    b2r = b2.reshape(1, _D)
    blinr = blin.reshape(1, 1)

    xp = jnp.pad(x, ((0, _NP - _N), (0, 0)))

    sum1p, cntp = _sc_agg(xp, src3, dst3, with_cnt=True)
    h1 = _tc_layer(sum1p, cntp, xp, W1l, b1r, W1r)
    sum2p = _sc_agg(h1, src3, dst3, with_cnt=False)
    y = _tc_layer_head(sum2p, cntp, h1, W2l, b2r, W2r, Wlin, blinr)
    return y[:_N, 0]


# speedup vs baseline: 3.0264x; 1.0619x over previous
import functools

import jax
import jax.numpy as jnp
from jax import lax
from jax.experimental import pallas as pl
from jax.experimental.pallas import tpu as pltpu
from jax.experimental.pallas import tpu_sc as plsc

_N = 10000
_NP = 10240
_E = 320000
_D = 128
_NC = 2
_NS = 16
_NW = _NC * _NS
_EP = 327680
_EPW = _EP // _NW
_C = 128
_NJ = _EPW // _C
_SL = 4
_SLB = 20
_NSL = _NJ // _SLB
_NWIN = _SLB // _SL
_RPT = _NP // _NS
_ZR = 8
_CW = 16


def _sc_agg(table, src3, dst3, with_cnt):
    outs = [jax.ShapeDtypeStruct((_NC, _NP, _D), jnp.float32)]
    if with_cnt:
        outs.append(jax.ShapeDtypeStruct((_NC, _NP, _D), jnp.float32))
    scratch = [
        pltpu.VMEM((_SLB, _C), jnp.int32),
        pltpu.VMEM((_SLB, _C), jnp.int32),
        pltpu.VMEM((2, _C, _D), jnp.float32),
        pltpu.VMEM((_ZR, _D), jnp.float32),
        pltpu.VMEM_SHARED((_NP, _D), jnp.float32),
        pltpu.SemaphoreType.DMA,
        pltpu.SemaphoreType.DMA,
        pltpu.SemaphoreType.DMA,
        pltpu.SemaphoreType.DMA,
    ]

    mesh = plsc.VectorSubcoreMesh(core_axis_name="c", subcore_axis_name="s")

    @functools.partial(pl.kernel, out_type=tuple(outs), mesh=mesh,
                       scratch_types=scratch)
    def k(table_h, src_h, dst_h, *refs):
        if with_cnt:
            (out_sum, out_cnt, src_v, dst_v, rows_v, zero_v, acc_s,
             gs0, gs1, ss0, ss1) = refs
        else:
            (out_sum, src_v, dst_v, rows_v, zero_v, acc_s,
             gs0, gs1, ss0, ss1) = refs
        gsem = (gs0, gs1)
        ssem = (ss0, ss1)

        core = lax.axis_index("c")
        sub = lax.axis_index("s")
        worker = core * _NS + sub
        base = sub * _RPT

        z16 = jnp.zeros((16,), jnp.float32)
        for i in range(_ZR):
            for jj in range(_D // 16):
                zero_v[i, pl.ds(jj * 16, 16)] = z16

        def zbody(i, _):
            pltpu.sync_copy(zero_v, acc_s.at[pl.ds(base + i * _ZR, _ZR)])
            return 0
        lax.fori_loop(0, _RPT // _ZR, zbody, 0)

        plsc.subcore_barrier()

        if with_cnt:
            one16 = jnp.ones((16,), jnp.float32)
            for i in range(_C):
                for jj in range(_D // 16):
                    rows_v[0, i, pl.ds(jj * 16, 16)] = one16

            def cbody(sl, _):
                pltpu.sync_copy(dst_h.at[worker, sl], dst_v)

                def cwin(w, _):
                    descs = [
                        pltpu.async_copy(rows_v.at[0],
                                         acc_s.at[dst_v.at[w * _SL + j]],
                                         ssem[j % 2], add=True)
                        for j in range(_SL)
                    ]
                    for d in descs:
                        d.wait()
                    return 0
                lax.fori_loop(0, _NWIN, cwin, 0)
                return 0
            lax.fori_loop(0, _NSL, cbody, 0)

            plsc.subcore_barrier()
            pltpu.sync_copy(acc_s.at[pl.ds(base, _RPT)],
                            out_cnt.at[core, pl.ds(base, _RPT)])
            lax.fori_loop(0, _RPT // _ZR, zbody, 0)
            plsc.subcore_barrier()

        def ebody(sl, _):
            pltpu.sync_copy(src_h.at[worker, sl], src_v)
            pltpu.sync_copy(dst_h.at[worker, sl], dst_v)
            pltpu.async_copy(table_h.at[src_v.at[0]], rows_v.at[0], gsem[0])

            def ewin(w, _):
                b = w * _SL
                for c in range(_SL):
                    j = b + c
                    p = c % 2
                    q = (c + 1) % 2

                    @pl.when(j >= 1)
                    def _():
                        pltpu.make_async_copy(
                            rows_v.at[q], acc_s.at[dst_v.at[j - 1]],
                            ssem[q]).wait()

                    @pl.when(j + 1 < _SLB)
                    def _():
                        pltpu.async_copy(table_h.at[src_v.at[j + 1]],
                                         rows_v.at[q], gsem[q])
                    pltpu.make_async_copy(table_h.at[src_v.at[j]],
                                          rows_v.at[p], gsem[p]).wait()
                    pltpu.async_copy(rows_v.at[p], acc_s.at[dst_v.at[j]],
                                     ssem[p], add=True)
                return 0
            lax.fori_loop(0, _NWIN, ewin, 0)
            pltpu.make_async_copy(rows_v.at[1],
                                  acc_s.at[dst_v.at[_SLB - 1]],
                                  ssem[1]).wait()
            return 0
        lax.fori_loop(0, _NSL, ebody, 0)

        plsc.subcore_barrier()

        pltpu.sync_copy(acc_s.at[pl.ds(base, _RPT)],
                        out_sum.at[core, pl.ds(base, _RPT)])

    res = k(table, src3, dst3)
    return res if with_cnt else res[0]


_R = 1024


def _tc_layer(sum_p, cnt_p, xin, Wl, b, Wr):
    def body(sum_ref, cnt_ref, x_ref, wl_ref, b_ref, wr_ref, o_ref):
        s = sum_ref[0] + sum_ref[1]
        c = cnt_ref[0, :, 0:1] + cnt_ref[1, :, 0:1]
        mean = s / jnp.maximum(c, 1.0)
        h = (jnp.dot(mean, wl_ref[...], preferred_element_type=jnp.float32)
             + jnp.dot(x_ref[...], wr_ref[...],
                       preferred_element_type=jnp.float32)
             + b_ref[...])
        o_ref[...] = jnp.maximum(h, 0.0)

    return pl.pallas_call(
        body,
        grid=(_NP // _R,),
        in_specs=[
            pl.BlockSpec((2, _R, _D), lambda i: (0, i, 0)),
            pl.BlockSpec((2, _R, _D), lambda i: (0, i, 0)),
            pl.BlockSpec((_R, _D), lambda i: (i, 0)),
            pl.BlockSpec((_D, _D), lambda i: (0, 0)),
            pl.BlockSpec((1, _D), lambda i: (0, 0)),
            pl.BlockSpec((_D, _D), lambda i: (0, 0)),
        ],
        out_specs=pl.BlockSpec((_R, _D), lambda i: (i, 0)),
        out_shape=jax.ShapeDtypeStruct((_NP, _D), jnp.float32),
    )(sum_p, cnt_p, xin, Wl, b, Wr)


def _tc_layer_head(sum_p, cnt_p, hin, Wl, b, Wr, Wlin, blin):
    def body(sum_ref, cnt_ref, h_ref, wl_ref, b_ref, wr_ref, wlin_ref,
             blin_ref, o_ref):
        s = sum_ref[0] + sum_ref[1]
        c = cnt_ref[0, :, 0:1] + cnt_ref[1, :, 0:1]
        mean = s / jnp.maximum(c, 1.0)
        h = (jnp.dot(mean, wl_ref[...], preferred_element_type=jnp.float32)
             + jnp.dot(h_ref[...], wr_ref[...],
                       preferred_element_type=jnp.float32)
             + b_ref[...])
        h = jnp.maximum(h, 0.0)
        o_ref[...] = (jnp.dot(h, wlin_ref[...],
                              preferred_element_type=jnp.float32)
                      + blin_ref[0, 0])

    return pl.pallas_call(
        body,
        grid=(_NP // _R,),
        in_specs=[
            pl.BlockSpec((2, _R, _D), lambda i: (0, i, 0)),
            pl.BlockSpec((2, _R, _D), lambda i: (0, i, 0)),
            pl.BlockSpec((_R, _D), lambda i: (i, 0)),
            pl.BlockSpec((_D, _D), lambda i: (0, 0)),
            pl.BlockSpec((1, _D), lambda i: (0, 0)),
            pl.BlockSpec((_D, _D), lambda i: (0, 0)),
            pl.BlockSpec((_D, 1), lambda i: (0, 0)),
            pl.BlockSpec((1, 1), lambda i: (0, 0)),
        ],
        out_specs=pl.BlockSpec((_R, 1), lambda i: (i, 0)),
        out_shape=jax.ShapeDtypeStruct((_NP, 1), jnp.float32),
    )(sum_p, cnt_p, hin, Wl, b, Wr, Wlin, blin)


def kernel(x, edge_index, W1l, b1, W1r, W2l, b2, W2r, Wlin, blin):
    pad = _N + (jnp.arange(_EP - _E, dtype=jnp.int32) % (_NP - _N))
    src3 = jnp.concatenate([edge_index[0].astype(jnp.int32), pad]
                           ).reshape(_NW, _NSL, _SLB, _C)
    dst3 = jnp.concatenate([edge_index[1].astype(jnp.int32), pad]
                           ).reshape(_NW, _NSL, _SLB, _C)
    b1r = b1.reshape(1, _D)
    b2r = b2.reshape(1, _D)
    blinr = blin.reshape(1, 1)

    xp = jnp.pad(x, ((0, _NP - _N), (0, 0)))

    sum1p, cntp = _sc_agg(xp, src3, dst3, with_cnt=True)
    h1 = _tc_layer(sum1p, cntp, xp, W1l, b1r, W1r)
    sum2p = _sc_agg(h1, src3, dst3, with_cnt=False)
    y = _tc_layer_head(sum2p, cntp, h1, W2l, b2r, W2r, Wlin, blinr)
    return y[:_N, 0]


# speedup vs baseline: 3.1315x; 1.0347x over previous
import functools

import jax
import jax.numpy as jnp
from jax import lax
from jax.experimental import pallas as pl
from jax.experimental.pallas import tpu as pltpu
from jax.experimental.pallas import tpu_sc as plsc

_N = 10000
_NP = 10240
_E = 320000
_D = 128
_NC = 2
_NS = 16
_NW = _NC * _NS
_EP = 327680
_EPW = _EP // _NW
_C = 128
_NJ = _EPW // _C
_SL = 4
_SLB = 20
_NSL = _NJ // _SLB
_NWIN = _SLB // _SL
_RPT = _NP // _NS
_ZR = 32
_CW = 16


def _sc_agg(table, src3, dst3, with_cnt):
    outs = [jax.ShapeDtypeStruct((_NC, _NP, _D), jnp.float32)]
    if with_cnt:
        outs.append(jax.ShapeDtypeStruct((_NC, _NP, _D), jnp.float32))
    scratch = [
        pltpu.VMEM((_SLB, _C), jnp.int32),
        pltpu.VMEM((_SLB, _C), jnp.int32),
        pltpu.VMEM((2, _C, _D), jnp.float32),
        pltpu.VMEM((_ZR, _D), jnp.float32),
        pltpu.VMEM_SHARED((_NP, _D), jnp.float32),
        pltpu.SemaphoreType.DMA,
        pltpu.SemaphoreType.DMA,
        pltpu.SemaphoreType.DMA,
        pltpu.SemaphoreType.DMA,
    ]

    mesh = plsc.VectorSubcoreMesh(core_axis_name="c", subcore_axis_name="s")

    @functools.partial(pl.kernel, out_type=tuple(outs), mesh=mesh,
                       scratch_types=scratch)
    def k(table_h, src_h, dst_h, *refs):
        if with_cnt:
            (out_sum, out_cnt, src_v, dst_v, rows_v, zero_v, acc_s,
             gs0, gs1, ss0, ss1) = refs
        else:
            (out_sum, src_v, dst_v, rows_v, zero_v, acc_s,
             gs0, gs1, ss0, ss1) = refs
        gsem = (gs0, gs1)
        ssem = (ss0, ss1)

        core = lax.axis_index("c")
        sub = lax.axis_index("s")
        worker = core * _NS + sub
        base = sub * _RPT

        z16 = jnp.zeros((16,), jnp.float32)
        for i in range(_ZR):
            for jj in range(_D // 16):
                zero_v[i, pl.ds(jj * 16, 16)] = z16

        def zbody(i, _):
            pltpu.sync_copy(zero_v, acc_s.at[pl.ds(base + i * _ZR, _ZR)])
            return 0
        lax.fori_loop(0, _RPT // _ZR, zbody, 0)

        plsc.subcore_barrier()

        if with_cnt:
            one16 = jnp.ones((16,), jnp.float32)
            for i in range(_C):
                for jj in range(_D // 16):
                    rows_v[0, i, pl.ds(jj * 16, 16)] = one16

            def cbody(sl, _):
                pltpu.sync_copy(dst_h.at[worker, sl], dst_v)

                def cwin(w, _):
                    descs = [
                        pltpu.async_copy(rows_v.at[0],
                                         acc_s.at[dst_v.at[w * _SL + j]],
                                         ssem[j % 2], add=True)
                        for j in range(_SL)
                    ]
                    for d in descs:
                        d.wait()
                    return 0
                lax.fori_loop(0, _NWIN, cwin, 0)
                return 0
            lax.fori_loop(0, _NSL, cbody, 0)

            plsc.subcore_barrier()
            pltpu.sync_copy(acc_s.at[pl.ds(base, _RPT)],
                            out_cnt.at[core, pl.ds(base, _RPT)])
            plsc.subcore_barrier()

        def ebody(sl, _):
            pltpu.sync_copy(src_h.at[worker, sl], src_v)
            pltpu.sync_copy(dst_h.at[worker, sl], dst_v)
            pltpu.async_copy(table_h.at[src_v.at[0]], rows_v.at[0], gsem[0])

            def ewin(w, _):
                b = w * _SL
                for c in range(_SL):
                    j = b + c
                    p = c % 2
                    q = (c + 1) % 2

                    @pl.when(j >= 1)
                    def _():
                        pltpu.make_async_copy(
                            rows_v.at[q], acc_s.at[dst_v.at[j - 1]],
                            ssem[q]).wait()

                    @pl.when(j + 1 < _SLB)
                    def _():
                        pltpu.async_copy(table_h.at[src_v.at[j + 1]],
                                         rows_v.at[q], gsem[q])
                    pltpu.make_async_copy(table_h.at[src_v.at[j]],
                                          rows_v.at[p], gsem[p]).wait()
                    pltpu.async_copy(rows_v.at[p], acc_s.at[dst_v.at[j]],
                                     ssem[p], add=True)
                return 0
            lax.fori_loop(0, _NWIN, ewin, 0)
            pltpu.make_async_copy(rows_v.at[1],
                                  acc_s.at[dst_v.at[_SLB - 1]],
                                  ssem[1]).wait()
            return 0
        lax.fori_loop(0, _NSL, ebody, 0)

        plsc.subcore_barrier()

        pltpu.sync_copy(acc_s.at[pl.ds(base, _RPT)],
                        out_sum.at[core, pl.ds(base, _RPT)])

    res = k(table, src3, dst3)
    return res if with_cnt else res[0]


_R = 1024


def _tc_layer(sum_p, cnt_p, xin, Wl, b, Wr):
    def body(sum_ref, cnt_ref, x_ref, wl_ref, b_ref, wr_ref, o_ref):
        s = (sum_ref[0] + sum_ref[1]) - (cnt_ref[0] + cnt_ref[1])
        c = cnt_ref[0, :, 0:1] + cnt_ref[1, :, 0:1]
        mean = s / jnp.maximum(c, 1.0)
        h = (jnp.dot(mean, wl_ref[...], preferred_element_type=jnp.float32)
             + jnp.dot(x_ref[...], wr_ref[...],
                       preferred_element_type=jnp.float32)
             + b_ref[...])
        o_ref[...] = jnp.maximum(h, 0.0)

    return pl.pallas_call(
        body,
        grid=(_NP // _R,),
        in_specs=[
            pl.BlockSpec((2, _R, _D), lambda i: (0, i, 0)),
            pl.BlockSpec((2, _R, _D), lambda i: (0, i, 0)),
            pl.BlockSpec((_R, _D), lambda i: (i, 0)),
            pl.BlockSpec((_D, _D), lambda i: (0, 0)),
            pl.BlockSpec((1, _D), lambda i: (0, 0)),
            pl.BlockSpec((_D, _D), lambda i: (0, 0)),
        ],
        out_specs=pl.BlockSpec((_R, _D), lambda i: (i, 0)),
        out_shape=jax.ShapeDtypeStruct((_NP, _D), jnp.float32),
    )(sum_p, cnt_p, xin, Wl, b, Wr)


def _tc_layer_head(sum_p, cnt_p, hin, Wl, b, Wr, Wlin, blin):
    def body(sum_ref, cnt_ref, h_ref, wl_ref, b_ref, wr_ref, wlin_ref,
             blin_ref, o_ref):
        s = sum_ref[0] + sum_ref[1]
        c = cnt_ref[0, :, 0:1] + cnt_ref[1, :, 0:1]
        mean = s / jnp.maximum(c, 1.0)
        h = (jnp.dot(mean, wl_ref[...], preferred_element_type=jnp.float32)
             + jnp.dot(h_ref[...], wr_ref[...],
                       preferred_element_type=jnp.float32)
             + b_ref[...])
        h = jnp.maximum(h, 0.0)
        o_ref[...] = (jnp.dot(h, wlin_ref[...],
                              preferred_element_type=jnp.float32)
                      + blin_ref[0, 0])

    return pl.pallas_call(
        body,
        grid=(_NP // _R,),
        in_specs=[
            pl.BlockSpec((2, _R, _D), lambda i: (0, i, 0)),
            pl.BlockSpec((2, _R, _D), lambda i: (0, i, 0)),
            pl.BlockSpec((_R, _D), lambda i: (i, 0)),
            pl.BlockSpec((_D, _D), lambda i: (0, 0)),
            pl.BlockSpec((1, _D), lambda i: (0, 0)),
            pl.BlockSpec((_D, _D), lambda i: (0, 0)),
            pl.BlockSpec((_D, 1), lambda i: (0, 0)),
            pl.BlockSpec((1, 1), lambda i: (0, 0)),
        ],
        out_specs=pl.BlockSpec((_R, 1), lambda i: (i, 0)),
        out_shape=jax.ShapeDtypeStruct((_NP, 1), jnp.float32),
    )(sum_p, cnt_p, hin, Wl, b, Wr, Wlin, blin)


def kernel(x, edge_index, W1l, b1, W1r, W2l, b2, W2r, Wlin, blin):
    pad = _N + (jnp.arange(_EP - _E, dtype=jnp.int32) % (_NP - _N))
    src3 = jnp.concatenate([edge_index[0].astype(jnp.int32), pad]
                           ).reshape(_NW, _NSL, _SLB, _C)
    dst3 = jnp.concatenate([edge_index[1].astype(jnp.int32), pad]
                           ).reshape(_NW, _NSL, _SLB, _C)
    b1r = b1.reshape(1, _D)
    b2r = b2.reshape(1, _D)
    blinr = blin.reshape(1, 1)

    xp = jnp.pad(x, ((0, _NP - _N), (0, 0)))

    sum1p, cntp = _sc_agg(xp, src3, dst3, with_cnt=True)
    h1 = _tc_layer(sum1p, cntp, xp, W1l, b1r, W1r)
    sum2p = _sc_agg(h1, src3, dst3, with_cnt=False)
    y = _tc_layer_head(sum2p, cntp, h1, W2l, b2r, W2r, Wlin, blinr)
    return y[:_N, 0]
